# baseline (speedup 1.0000x reference)
import dataclasses
import functools

import jax
import jax.numpy as jnp
from jax import lax
from jax.experimental import pallas as pl
from jax.experimental.pallas import tpu as pltpu
from jax.experimental.pallas import tpu_sc as plsc

N = 10000
E = 320000
D = 128
OUT = 4

NPAD = 10240
EPAD = 327680
NCORES = 2
NSUB = 16
NW = NCORES * NSUB
E_PER_W = EPAD // NW
CHUNK = 64
N_CHUNKS = E_PER_W // CHUNK
ROWS_PER_SUB = NPAD // NSUB
ZROWS = 64


def _sc_mesh():
    return plsc.VectorSubcoreMesh(core_axis_name="c", subcore_axis_name="s")


def _sc_compiler_params():
    cp = pltpu.CompilerParams()
    if "needs_layout_passes" in pltpu.CompilerParams.__dataclass_fields__:
        cp = dataclasses.replace(cp, needs_layout_passes=False)
    return cp


CROWS = NPAD // D
CSUBS = CROWS // 8
NGROUP = CHUNK // 16
IB = 1024
NIB = E_PER_W // IB
NC2 = IB // CHUNK
RING = 4


def _make_segment_sum(with_counts):
    out_type = jax.ShapeDtypeStruct((NCORES * NPAD, D), jnp.float32)
    if with_counts:
        out_type = (out_type,
                    jax.ShapeDtypeStruct((NCORES * CROWS, D), jnp.float32))
    scratch = [
        pltpu.VMEM_SHARED((NPAD, D), jnp.float32),
        pltpu.VMEM((IB,), jnp.int32),
        pltpu.VMEM((IB,), jnp.int32),
        pltpu.VMEM((CHUNK, D), jnp.float32),
        pltpu.VMEM((CHUNK, D), jnp.float32),
        pltpu.VMEM((CHUNK, D), jnp.float32),
        pltpu.VMEM((CHUNK, D), jnp.float32),
        pltpu.VMEM((CHUNK,), jnp.int32),
        pltpu.SemaphoreType.DMA,
        pltpu.SemaphoreType.DMA,
        pltpu.SemaphoreType.DMA,
        pltpu.SemaphoreType.DMA,
    ]
    if with_counts:
        scratch += [
            pltpu.VMEM_SHARED((CROWS, D), jnp.float32),
            pltpu.VMEM((CHUNK, D), jnp.float32),
            pltpu.VMEM((CHUNK,), jnp.int32),
        ]

    def body(refs):
        if with_counts:
            (x_hbm, src_hbm, dst_hbm, zeros_hbm, sum_out, cnt_out,
             acc_sh, sblk_v, dblk_v, rows0_v, rows1_v, rows2_v, rows3_v,
             dstc_v, sem0, sem1, sem2, sem3, cnt_sh, crows_v, cidx_v) = refs
        else:
            (x_hbm, src_hbm, dst_hbm, zeros_hbm, sum_out,
             acc_sh, sblk_v, dblk_v, rows0_v, rows1_v, rows2_v, rows3_v,
             dstc_v, sem0, sem1, sem2, sem3) = refs
        cid = lax.axis_index("c")
        sid = lax.axis_index("s")
        wid = sid * NCORES + cid
        it16 = lax.iota(jnp.int32, 16)
        ones16 = jnp.ones((16,), jnp.float32)
        zeros16 = jnp.zeros((16,), jnp.float32)

        pltpu.sync_copy(zeros_hbm, rows0_v)
        if with_counts:
            pltpu.sync_copy(zeros_hbm, crows_v)

        row0 = sid * ROWS_PER_SUB

        @pl.loop(0, ROWS_PER_SUB, step=ZROWS)
        def _(r):
            pltpu.sync_copy(rows0_v, acc_sh.at[pl.ds(row0 + r, ZROWS)])

        if with_counts:
            @pl.when(sid < CSUBS)
            def _():
                pltpu.sync_copy(rows0_v.at[pl.ds(0, 8)],
                                cnt_sh.at[pl.ds(sid * 8, 8)])

        plsc.subcore_barrier()

        bufs = (rows0_v, rows1_v, rows2_v, rows3_v)
        sems = (sem0, sem1, sem2, sem3)

        def g_issue(c, buf, sem):
            pltpu.async_copy(
                x_hbm.at[sblk_v.at[pl.ds(c * CHUNK, CHUNK)]], buf, sem)

        def g_wait(buf, sem):
            pltpu.make_async_copy(
                x_hbm.at[sblk_v.at[pl.ds(0, CHUNK)]], buf, sem).wait()

        def process(c, buf):
            for g in range(NGROUP):
                d16 = dblk_v[pl.ds(c * CHUNK + g * 16, 16)]
                dstc_v[pl.ds(g * 16, 16)] = d16
                if with_counts:
                    col = lax.bitwise_and(d16, D - 1)
                    row = it16 + (g * 16)
                    plsc.store_scatter(crows_v, [row, col], ones16)
                    cidx_v[pl.ds(g * 16, 16)] = lax.shift_right_logical(d16, 7)
            pltpu.sync_copy(buf, acc_sh.at[dstc_v], add=True)
            if with_counts:
                pltpu.sync_copy(crows_v, cnt_sh.at[cidx_v], add=True)
                for g in range(NGROUP):
                    d16 = dblk_v[pl.ds(c * CHUNK + g * 16, 16)]
                    col = lax.bitwise_and(d16, D - 1)
                    row = it16 + (g * 16)
                    plsc.store_scatter(crows_v, [row, col], zeros16)

        ebase = wid * E_PER_W

        @pl.loop(0, NIB)
        def _(bi):
            boff = ebase + bi * IB
            pltpu.sync_copy(src_hbm.at[pl.ds(boff, IB)], sblk_v)
            pltpu.sync_copy(dst_hbm.at[pl.ds(boff, IB)], dblk_v)

            for j in range(RING - 1):
                g_issue(j, bufs[j], sems[j])

            @pl.loop(0, NC2 - RING, step=RING)
            def _(c0):
                for j in range(RING):
                    g_wait(bufs[j], sems[j])
                    process(c0 + j, bufs[j])
                    g_issue(c0 + j + RING - 1,
                            bufs[(j + RING - 1) % RING], sems[(j + RING - 1) % RING])

            c0 = NC2 - RING
            for j in range(RING):
                g_wait(bufs[j], sems[j])
                process(c0 + j, bufs[j])
                if c0 + j + RING - 1 < NC2:
                    g_issue(c0 + j + RING - 1,
                            bufs[(j + RING - 1) % RING], sems[(j + RING - 1) % RING])

        plsc.subcore_barrier()

        out0 = cid * NPAD + row0

        @pl.loop(0, ROWS_PER_SUB, step=ZROWS)
        def _(r):
            pltpu.sync_copy(acc_sh.at[pl.ds(row0 + r, ZROWS)],
                            sum_out.at[pl.ds(out0 + r, ZROWS)])

        if with_counts:
            @pl.when(sid < CSUBS)
            def _():
                pltpu.sync_copy(
                    cnt_sh.at[pl.ds(sid * 8, 8)],
                    cnt_out.at[pl.ds(cid * CROWS + sid * 8, 8)])

    @functools.partial(
        pl.kernel,
        mesh=_sc_mesh(),
        out_type=out_type,
        compiler_params=_sc_compiler_params(),
        scratch_types=scratch,
    )
    def k(*refs):
        body(refs)

    def run(x_pad, src, dst):
        return k(x_pad, src, dst, jnp.zeros((ZROWS, D), jnp.float32))

    return run


def _segment_sum_counts(x_pad, src, dst):
    return _make_segment_sum(True)(x_pad, src, dst)


def _segment_sum_only(x_pad, src, dst):
    return _make_segment_sum(False)(x_pad, src, dst)


BLK = 512
GRID = NPAD // BLK


def _tc_layer_body(p0, p1, c0, c1, x, wl, wr, bl, g, b):
    s = p0[...] + p1[...]
    c = c0[...] + c1[...]
    mean = s / jnp.maximum(c, 1.0)
    h = lax.dot_general(mean, wl[...], (((1,), (1,)), ((), ())),
                        precision=lax.Precision.HIGHEST,
                        preferred_element_type=jnp.float32)
    h = h + lax.dot_general(x[...], wr[...], (((1,), (1,)), ((), ())),
                            precision=lax.Precision.HIGHEST,
                            preferred_element_type=jnp.float32)
    h = h + bl[...]
    m = jnp.mean(h, axis=1, keepdims=True)
    v = jnp.mean((h - m) * (h - m), axis=1, keepdims=True)
    h = (h - m) * lax.rsqrt(v + 1e-5) * g[...] + b[...]
    return jnp.maximum(h, 0.0)


def _tc_layer(sums, cnt2, x_pad, Wl, bl, Wr, g, b):
    def body(p0_r, p1_r, c0_r, c1_r, x_r, wl_r, wr_r, bl_r, g_r, b_r, o_r):
        o_r[...] = _tc_layer_body(p0_r, p1_r, c0_r, c1_r, x_r,
                                  wl_r, wr_r, bl_r, g_r, b_r)

    row_spec = pl.BlockSpec((BLK, D), lambda i: (i, 0))
    w_spec = pl.BlockSpec((D, D), lambda i: (0, 0))
    v_spec = pl.BlockSpec((1, D), lambda i: (0, 0))
    return pl.pallas_call(
        body,
        grid=(GRID,),
        in_specs=[
            pl.BlockSpec((BLK, D), lambda i: (i, 0)),
            pl.BlockSpec((BLK, D), lambda i: (i + GRID, 0)),
            pl.BlockSpec((BLK, 1), lambda i: (i, 0)),
            pl.BlockSpec((BLK, 1), lambda i: (i + GRID, 0)),
            row_spec, w_spec, w_spec, v_spec, v_spec, v_spec,
        ],
        out_specs=row_spec,
        out_shape=jax.ShapeDtypeStruct((NPAD, D), jnp.float32),
    )(sums, sums, cnt2, cnt2, x_pad, Wl, Wr,
      bl.reshape(1, D), g.reshape(1, D), b.reshape(1, D))


def _tc_layer_head(sums, cnt2, x_pad, Wl, bl, Wr, g, b, WhP, bhP):
    def body(p0_r, p1_r, c0_r, c1_r, x_r, wl_r, wr_r, bl_r, g_r, b_r,
             wh_r, bh_r, o_r):
        h = _tc_layer_body(p0_r, p1_r, c0_r, c1_r, x_r,
                           wl_r, wr_r, bl_r, g_r, b_r)
        o_r[...] = lax.dot_general(h, wh_r[...], (((1,), (1,)), ((), ())),
                                   precision=lax.Precision.HIGHEST,
                                   preferred_element_type=jnp.float32) + bh_r[...]

    row_spec = pl.BlockSpec((BLK, D), lambda i: (i, 0))
    w_spec = pl.BlockSpec((D, D), lambda i: (0, 0))
    v_spec = pl.BlockSpec((1, D), lambda i: (0, 0))
    return pl.pallas_call(
        body,
        grid=(GRID,),
        in_specs=[
            pl.BlockSpec((BLK, D), lambda i: (i, 0)),
            pl.BlockSpec((BLK, D), lambda i: (i + GRID, 0)),
            pl.BlockSpec((BLK, 1), lambda i: (i, 0)),
            pl.BlockSpec((BLK, 1), lambda i: (i + GRID, 0)),
            row_spec, w_spec, w_spec, v_spec, v_spec, v_spec,
            w_spec, v_spec,
        ],
        out_specs=row_spec,
        out_shape=jax.ShapeDtypeStruct((NPAD, D), jnp.float32),
    )(sums, sums, cnt2, cnt2, x_pad, Wl, Wr,
      bl.reshape(1, D), g.reshape(1, D), b.reshape(1, D), WhP, bhP)


def kernel(x, edge_index, W1l, b1l, W1r, ln1_g, ln1_b,
           W2l, b2l, W2r, ln2_g, ln2_b, Wh, bh):
    src = jnp.concatenate([edge_index[0], jnp.zeros((EPAD - E,), jnp.int32)])
    dst = jnp.concatenate([edge_index[1], jnp.full((EPAD - E,), N, jnp.int32)])
    x_pad = jnp.pad(x, ((0, NPAD - N), (0, 0)))

    sums1, cnts = _segment_sum_counts(x_pad, src, dst)
    cnt2 = cnts.reshape(NCORES * NPAD, 1)
    h1 = _tc_layer(sums1, cnt2, x_pad, W1l, b1l, W1r, ln1_g, ln1_b)
    sums2 = _segment_sum_only(h1, src, dst)
    WhP = jnp.zeros((D, D), jnp.float32).at[:OUT].set(Wh)
    bhP = jnp.zeros((1, D), jnp.float32).at[0, :OUT].set(bh)
    out = _tc_layer_head(sums2, cnt2, h1, W2l, b2l, W2r, ln2_g, ln2_b, WhP, bhP)
    return out[:N, :OUT]


# speedup vs baseline: 1.0006x; 1.0006x over previous
import dataclasses
import functools

import jax
import jax.numpy as jnp
from jax import lax
from jax.experimental import pallas as pl
from jax.experimental.pallas import tpu as pltpu
from jax.experimental.pallas import tpu_sc as plsc

N = 10000
E = 320000
D = 128
OUT = 4

NPAD = 10240
EPAD = 327680
NCORES = 2
NSUB = 16
NW = NCORES * NSUB
E_PER_W = EPAD // NW
CHUNK = 64
N_CHUNKS = E_PER_W // CHUNK
ROWS_PER_SUB = NPAD // NSUB
ZROWS = 64


def _sc_mesh():
    return plsc.VectorSubcoreMesh(core_axis_name="c", subcore_axis_name="s")


def _sc_compiler_params():
    cp = pltpu.CompilerParams()
    if "needs_layout_passes" in pltpu.CompilerParams.__dataclass_fields__:
        cp = dataclasses.replace(cp, needs_layout_passes=False)
    return cp


CROWS = NPAD // D
CSUBS = CROWS // 8
NGROUP = CHUNK // 16
IB = 1024
NIB = E_PER_W // IB
NC2 = IB // CHUNK
RING = 4


def _make_segment_sum(with_counts):
    out_type = jax.ShapeDtypeStruct((NCORES * NPAD, D), jnp.float32)
    if with_counts:
        out_type = (out_type,
                    jax.ShapeDtypeStruct((NCORES * CROWS, D), jnp.float32))
    scratch = [
        pltpu.VMEM_SHARED((NPAD, D), jnp.float32),
        pltpu.VMEM((IB,), jnp.int32),
        pltpu.VMEM((IB,), jnp.int32),
        pltpu.VMEM((CHUNK, D), jnp.float32),
        pltpu.VMEM((CHUNK, D), jnp.float32),
        pltpu.VMEM((CHUNK, D), jnp.float32),
        pltpu.VMEM((CHUNK, D), jnp.float32),
        pltpu.VMEM((CHUNK,), jnp.int32),
        pltpu.SemaphoreType.DMA,
        pltpu.SemaphoreType.DMA,
        pltpu.SemaphoreType.DMA,
        pltpu.SemaphoreType.DMA,
    ]
    if with_counts:
        scratch += [
            pltpu.VMEM_SHARED((CROWS, D), jnp.float32),
            pltpu.VMEM((CHUNK, D), jnp.float32),
            pltpu.VMEM((CHUNK,), jnp.int32),
        ]

    def body(refs):
        if with_counts:
            (x_hbm, src_hbm, dst_hbm, zeros_hbm, sum_out, cnt_out,
             acc_sh, sblk_v, dblk_v, rows0_v, rows1_v, rows2_v, rows3_v,
             dstc_v, sem0, sem1, sem2, sem3, cnt_sh, crows_v, cidx_v) = refs
        else:
            (x_hbm, src_hbm, dst_hbm, zeros_hbm, sum_out,
             acc_sh, sblk_v, dblk_v, rows0_v, rows1_v, rows2_v, rows3_v,
             dstc_v, sem0, sem1, sem2, sem3) = refs
        cid = lax.axis_index("c")
        sid = lax.axis_index("s")
        wid = sid * NCORES + cid
        it16 = lax.iota(jnp.int32, 16)
        ones16 = jnp.ones((16,), jnp.float32)
        zeros16 = jnp.zeros((16,), jnp.float32)

        pltpu.sync_copy(zeros_hbm, rows0_v)
        if with_counts:
            pltpu.sync_copy(zeros_hbm, crows_v)

        row0 = sid * ROWS_PER_SUB

        @pl.loop(0, ROWS_PER_SUB, step=ZROWS)
        def _(r):
            pltpu.sync_copy(rows0_v, acc_sh.at[pl.ds(row0 + r, ZROWS)])

        if with_counts:
            @pl.when(sid < CSUBS)
            def _():
                pltpu.sync_copy(rows0_v.at[pl.ds(0, 8)],
                                cnt_sh.at[pl.ds(sid * 8, 8)])

        plsc.subcore_barrier()

        bufs = (rows0_v, rows1_v, rows2_v, rows3_v)
        sems = (sem0, sem1, sem2, sem3)

        def g_issue(c, buf, sem):
            pltpu.async_copy(
                x_hbm.at[sblk_v.at[pl.ds(c * CHUNK, CHUNK)]], buf, sem)

        def g_wait(buf, sem):
            pltpu.make_async_copy(
                x_hbm.at[sblk_v.at[pl.ds(0, CHUNK)]], buf, sem).wait()

        def process(c, buf):
            for g in range(NGROUP):
                d16 = dblk_v[pl.ds(c * CHUNK + g * 16, 16)]
                dstc_v[pl.ds(g * 16, 16)] = d16
                if with_counts:
                    col = lax.bitwise_and(d16, D - 1)
                    row = it16 + (g * 16)
                    plsc.store_scatter(crows_v, [row, col], ones16)
                    cidx_v[pl.ds(g * 16, 16)] = lax.shift_right_logical(d16, 7)
            pltpu.sync_copy(buf, acc_sh.at[dstc_v], add=True)
            if with_counts:
                pltpu.sync_copy(crows_v, cnt_sh.at[cidx_v], add=True)
                for g in range(NGROUP):
                    d16 = dblk_v[pl.ds(c * CHUNK + g * 16, 16)]
                    col = lax.bitwise_and(d16, D - 1)
                    row = it16 + (g * 16)
                    plsc.store_scatter(crows_v, [row, col], zeros16)

        ebase = wid * E_PER_W

        @pl.loop(0, NIB)
        def _(bi):
            boff = ebase + bi * IB
            pltpu.sync_copy(src_hbm.at[pl.ds(boff, IB)], sblk_v)
            pltpu.sync_copy(dst_hbm.at[pl.ds(boff, IB)], dblk_v)

            for j in range(RING - 1):
                g_issue(j, bufs[j], sems[j])

            @pl.loop(0, NC2 - RING, step=RING)
            def _(c0):
                for j in range(RING):
                    g_wait(bufs[j], sems[j])
                    process(c0 + j, bufs[j])
                    g_issue(c0 + j + RING - 1,
                            bufs[(j + RING - 1) % RING], sems[(j + RING - 1) % RING])

            c0 = NC2 - RING
            for j in range(RING):
                g_wait(bufs[j], sems[j])
                process(c0 + j, bufs[j])
                if c0 + j + RING - 1 < NC2:
                    g_issue(c0 + j + RING - 1,
                            bufs[(j + RING - 1) % RING], sems[(j + RING - 1) % RING])

        plsc.subcore_barrier()

        out0 = cid * NPAD + row0

        @pl.loop(0, ROWS_PER_SUB, step=ZROWS)
        def _(r):
            pltpu.sync_copy(acc_sh.at[pl.ds(row0 + r, ZROWS)],
                            sum_out.at[pl.ds(out0 + r, ZROWS)])

        if with_counts:
            @pl.when(sid < CSUBS)
            def _():
                pltpu.sync_copy(
                    cnt_sh.at[pl.ds(sid * 8, 8)],
                    cnt_out.at[pl.ds(cid * CROWS + sid * 8, 8)])

    @functools.partial(
        pl.kernel,
        mesh=_sc_mesh(),
        out_type=out_type,
        compiler_params=_sc_compiler_params(),
        scratch_types=scratch,
    )
    def k(*refs):
        body(refs)

    def run(x_pad, src, dst):
        return k(x_pad, src, dst, jnp.zeros((ZROWS, D), jnp.float32))

    return run


def _segment_sum_counts(x_pad, src, dst):
    return _make_segment_sum(True)(x_pad, src, dst)


def _segment_sum_only(x_pad, src, dst):
    return _make_segment_sum(False)(x_pad, src, dst)


BLK = 512
GRID = NPAD // BLK


def _tc_layer_body(p0, p1, c0, c1, x, wl, wr, bl, g, b):
    s = p0[...] + p1[...]
    c = c0[...] + c1[...]
    mean = s / jnp.maximum(c, 1.0)
    h = lax.dot_general(mean, wl[...], (((1,), (1,)), ((), ())),
                        precision=lax.Precision.HIGHEST,
                        preferred_element_type=jnp.float32)
    h = h + lax.dot_general(x[...], wr[...], (((1,), (1,)), ((), ())),
                            precision=lax.Precision.HIGHEST,
                            preferred_element_type=jnp.float32)
    h = h + bl[...]
    m = jnp.mean(h, axis=1, keepdims=True)
    v = jnp.mean((h - m) * (h - m), axis=1, keepdims=True)
    h = (h - m) * lax.rsqrt(v + 1e-5) * g[...] + b[...]
    return jnp.maximum(h, 0.0)


def _tc_layer(sums, cnt2, x_pad, Wl, bl, Wr, g, b):
    def body(p0_r, p1_r, c0_r, c1_r, x_r, wl_r, wr_r, bl_r, g_r, b_r, o_r):
        o_r[...] = _tc_layer_body(p0_r, p1_r, c0_r, c1_r, x_r,
                                  wl_r, wr_r, bl_r, g_r, b_r)

    row_spec = pl.BlockSpec((BLK, D), lambda i: (i, 0))
    w_spec = pl.BlockSpec((D, D), lambda i: (0, 0))
    v_spec = pl.BlockSpec((1, D), lambda i: (0, 0))
    return pl.pallas_call(
        body,
        grid=(GRID,),
        in_specs=[
            pl.BlockSpec((BLK, D), lambda i: (i, 0)),
            pl.BlockSpec((BLK, D), lambda i: (i + GRID, 0)),
            pl.BlockSpec((BLK, 1), lambda i: (i, 0)),
            pl.BlockSpec((BLK, 1), lambda i: (i + GRID, 0)),
            row_spec, w_spec, w_spec, v_spec, v_spec, v_spec,
        ],
        out_specs=row_spec,
        out_shape=jax.ShapeDtypeStruct((NPAD, D), jnp.float32),
    )(sums, sums, cnt2, cnt2, x_pad, Wl, Wr,
      bl.reshape(1, D), g.reshape(1, D), b.reshape(1, D))


def _tc_layer_head(sums, cnt2, x_pad, Wl, bl, Wr, g, b, WhP, bhP):
    def body(p0_r, p1_r, c0_r, c1_r, x_r, wl_r, wr_r, bl_r, g_r, b_r,
             wh_r, bh_r, o_r):
        h = _tc_layer_body(p0_r, p1_r, c0_r, c1_r, x_r,
                           wl_r, wr_r, bl_r, g_r, b_r)
        o_r[...] = lax.dot_general(h, wh_r[...], (((1,), (1,)), ((), ())),
                                   precision=lax.Precision.HIGHEST,
                                   preferred_element_type=jnp.float32) + bh_r[...]

    row_spec = pl.BlockSpec((BLK, D), lambda i: (i, 0))
    w_spec = pl.BlockSpec((D, D), lambda i: (0, 0))
    v_spec = pl.BlockSpec((1, D), lambda i: (0, 0))
    return pl.pallas_call(
        body,
        grid=(GRID,),
        in_specs=[
            pl.BlockSpec((BLK, D), lambda i: (i, 0)),
            pl.BlockSpec((BLK, D), lambda i: (i + GRID, 0)),
            pl.BlockSpec((BLK, 1), lambda i: (i, 0)),
            pl.BlockSpec((BLK, 1), lambda i: (i + GRID, 0)),
            row_spec, w_spec, w_spec, v_spec, v_spec, v_spec,
            w_spec, v_spec,
        ],
        out_specs=row_spec,
        out_shape=jax.ShapeDtypeStruct((NPAD, D), jnp.float32),
    )(sums, sums, cnt2, cnt2, x_pad, Wl, Wr,
      bl.reshape(1, D), g.reshape(1, D), b.reshape(1, D), WhP, bhP)


def kernel(x, edge_index, W1l, b1l, W1r, ln1_g, ln1_b,
           W2l, b2l, W2r, ln2_g, ln2_b, Wh, bh):
    src = jnp.concatenate([edge_index[0], jnp.zeros((EPAD - E,), jnp.int32)])
    pad_dst = N + jnp.arange(EPAD - E, dtype=jnp.int32) % (NPAD - N)
    dst = jnp.concatenate([edge_index[1], pad_dst])
    x_pad = jnp.pad(x, ((0, NPAD - N), (0, 0)))

    sums1, cnts = _segment_sum_counts(x_pad, src, dst)
    cnt2 = cnts.reshape(NCORES * NPAD, 1)
    h1 = _tc_layer(sums1, cnt2, x_pad, W1l, b1l, W1r, ln1_g, ln1_b)
    sums2 = _segment_sum_only(h1, src, dst)
    WhP = jnp.zeros((D, D), jnp.float32).at[:OUT].set(Wh)
    bhP = jnp.zeros((1, D), jnp.float32).at[0, :OUT].set(bh)
    out = _tc_layer_head(sums2, cnt2, h1, W2l, b2l, W2r, ln2_g, ln2_b, WhP, bhP)
    return out[:N, :OUT]


# speedup vs baseline: 1.0977x; 1.0970x over previous
import dataclasses
import functools

import jax
import jax.numpy as jnp
from jax import lax
from jax.experimental import pallas as pl
from jax.experimental.pallas import tpu as pltpu
from jax.experimental.pallas import tpu_sc as plsc

N = 10000
E = 320000
D = 128
OUT = 4

NPAD = 10240
EPAD = 327680
NCORES = 2
NSUB = 16
NW = NCORES * NSUB
E_PER_W = EPAD // NW
CHUNK = 64
N_CHUNKS = E_PER_W // CHUNK
ROWS_PER_SUB = NPAD // NSUB
ZROWS = 64


def _sc_mesh():
    return plsc.VectorSubcoreMesh(core_axis_name="c", subcore_axis_name="s")


def _sc_compiler_params():
    cp = pltpu.CompilerParams()
    if "needs_layout_passes" in pltpu.CompilerParams.__dataclass_fields__:
        cp = dataclasses.replace(cp, needs_layout_passes=False)
    return cp


CROWS = NPAD // D
CSUBS = CROWS // 8
NGROUP = CHUNK // 16
IB = 1024
NIB = E_PER_W // IB
NC2 = IB // CHUNK
RING = 4


def _make_segment_sum(with_counts):
    out_type = jax.ShapeDtypeStruct((NCORES * NPAD, D), jnp.float32)
    if with_counts:
        out_type = (out_type,
                    jax.ShapeDtypeStruct((NCORES * CROWS, D), jnp.float32))
    scratch = [
        pltpu.VMEM_SHARED((NPAD, D), jnp.float32),
        pltpu.VMEM((IB,), jnp.int32),
        pltpu.VMEM((IB,), jnp.int32),
        pltpu.VMEM((CHUNK, D), jnp.float32),
        pltpu.VMEM((CHUNK, D), jnp.float32),
        pltpu.VMEM((CHUNK, D), jnp.float32),
        pltpu.VMEM((CHUNK, D), jnp.float32),
        pltpu.VMEM((CHUNK,), jnp.int32),
        pltpu.VMEM((CHUNK,), jnp.int32),
        pltpu.SemaphoreType.DMA,
        pltpu.SemaphoreType.DMA,
        pltpu.SemaphoreType.DMA,
        pltpu.SemaphoreType.DMA,
        pltpu.SemaphoreType.DMA,
    ]
    if with_counts:
        scratch += [
            pltpu.VMEM_SHARED((CROWS, D), jnp.float32),
            pltpu.VMEM((CHUNK, D), jnp.float32),
            pltpu.VMEM((CHUNK,), jnp.int32),
        ]

    def body(refs):
        if with_counts:
            (x_hbm, src_hbm, dst_hbm, zeros_hbm, tidx_hbm, sum_out, cnt_out,
             acc_sh, sblk_v, dblk_v, rows0_v, rows1_v, rows2_v, rows3_v,
             dstc0_v, dstc1_v, sem0, sem1, sem2, sem3, ssem,
             cnt_sh, crows_v, cidx_v) = refs
        else:
            (x_hbm, src_hbm, dst_hbm, zeros_hbm, tidx_hbm, sum_out,
             acc_sh, sblk_v, dblk_v, rows0_v, rows1_v, rows2_v, rows3_v,
             dstc0_v, dstc1_v, sem0, sem1, sem2, sem3, ssem) = refs
        cid = lax.axis_index("c")
        sid = lax.axis_index("s")
        wid = sid * NCORES + cid
        it16 = lax.iota(jnp.int32, 16)
        ones16 = jnp.ones((16,), jnp.float32)
        zeros16 = jnp.zeros((16,), jnp.float32)

        pltpu.sync_copy(zeros_hbm, rows0_v)
        if with_counts:
            pltpu.sync_copy(zeros_hbm, crows_v)

        row0 = sid * ROWS_PER_SUB

        @pl.loop(0, ROWS_PER_SUB, step=ZROWS)
        def _(r):
            pltpu.sync_copy(rows0_v, acc_sh.at[pl.ds(row0 + r, ZROWS)])

        if with_counts:
            @pl.when(sid < CSUBS)
            def _():
                pltpu.sync_copy(rows0_v.at[pl.ds(0, 8)],
                                cnt_sh.at[pl.ds(sid * 8, 8)])

        plsc.subcore_barrier()

        bufs = (rows0_v, rows1_v, rows2_v, rows3_v)
        sems = (sem0, sem1, sem2, sem3)
        dstcs = (dstc0_v, dstc1_v)

        def g_issue(c, buf, sem):
            pltpu.async_copy(
                x_hbm.at[sblk_v.at[pl.ds(c * CHUNK, CHUNK)]], buf, sem)

        def g_wait(buf, sem):
            pltpu.make_async_copy(
                x_hbm.at[sblk_v.at[pl.ds(0, CHUNK)]], buf, sem).wait()

        def s_wait():
            pltpu.make_async_copy(rows0_v, acc_sh.at[dstc0_v], ssem).wait()

        def process(c, buf, p):
            dstc = dstcs[p]
            for g in range(NGROUP):
                d16 = dblk_v[pl.ds(c * CHUNK + g * 16, 16)]
                dstc[pl.ds(g * 16, 16)] = d16
                if with_counts:
                    col = lax.bitwise_and(d16, D - 1)
                    row = it16 + (g * 16)
                    plsc.store_scatter(crows_v, [row, col], ones16)
                    cidx_v[pl.ds(g * 16, 16)] = lax.shift_right_logical(d16, 7)
            s_wait()
            pltpu.async_copy(buf, acc_sh.at[dstc], ssem, add=True)
            if with_counts:
                pltpu.sync_copy(crows_v, cnt_sh.at[cidx_v], add=True)
                for g in range(NGROUP):
                    d16 = dblk_v[pl.ds(c * CHUNK + g * 16, 16)]
                    col = lax.bitwise_and(d16, D - 1)
                    row = it16 + (g * 16)
                    plsc.store_scatter(crows_v, [row, col], zeros16)

        ebase = wid * E_PER_W

        pltpu.sync_copy(tidx_hbm, dstc0_v)
        pltpu.async_copy(rows0_v, acc_sh.at[dstc0_v], ssem, add=True)

        @pl.loop(0, NIB)
        def _(bi):
            boff = ebase + bi * IB
            pltpu.sync_copy(src_hbm.at[pl.ds(boff, IB)], sblk_v)
            pltpu.sync_copy(dst_hbm.at[pl.ds(boff, IB)], dblk_v)

            for j in range(RING - 1):
                g_issue(j, bufs[j], sems[j])

            @pl.loop(0, NC2 - RING, step=RING)
            def _(c0):
                for j in range(RING):
                    g_wait(bufs[j], sems[j])
                    process(c0 + j, bufs[j], j % 2)
                    g_issue(c0 + j + RING - 1,
                            bufs[(j + RING - 1) % RING], sems[(j + RING - 1) % RING])

            c0 = NC2 - RING
            for j in range(RING):
                g_wait(bufs[j], sems[j])
                process(c0 + j, bufs[j], j % 2)
                if c0 + j + RING - 1 < NC2:
                    g_issue(c0 + j + RING - 1,
                            bufs[(j + RING - 1) % RING], sems[(j + RING - 1) % RING])

        s_wait()
        plsc.subcore_barrier()

        out0 = cid * NPAD + row0

        @pl.loop(0, ROWS_PER_SUB, step=ZROWS)
        def _(r):
            pltpu.sync_copy(acc_sh.at[pl.ds(row0 + r, ZROWS)],
                            sum_out.at[pl.ds(out0 + r, ZROWS)])

        if with_counts:
            @pl.when(sid < CSUBS)
            def _():
                pltpu.sync_copy(
                    cnt_sh.at[pl.ds(sid * 8, 8)],
                    cnt_out.at[pl.ds(cid * CROWS + sid * 8, 8)])

    @functools.partial(
        pl.kernel,
        mesh=_sc_mesh(),
        out_type=out_type,
        compiler_params=_sc_compiler_params(),
        scratch_types=scratch,
    )
    def k(*refs):
        body(refs)

    def run(x_pad, src, dst):
        return k(x_pad, src, dst, jnp.zeros((ZROWS, D), jnp.float32),
                 jnp.full((CHUNK,), NPAD - 1, jnp.int32))

    return run


def _segment_sum_counts(x_pad, src, dst):
    return _make_segment_sum(True)(x_pad, src, dst)


def _segment_sum_only(x_pad, src, dst):
    return _make_segment_sum(False)(x_pad, src, dst)


BLK = 512
GRID = NPAD // BLK


def _tc_layer_body(p0, p1, c0, c1, x, wl, wr, bl, g, b):
    s = p0[...] + p1[...]
    c = c0[...] + c1[...]
    mean = s / jnp.maximum(c, 1.0)
    h = lax.dot_general(mean, wl[...], (((1,), (1,)), ((), ())),
                        precision=lax.Precision.HIGHEST,
                        preferred_element_type=jnp.float32)
    h = h + lax.dot_general(x[...], wr[...], (((1,), (1,)), ((), ())),
                            precision=lax.Precision.HIGHEST,
                            preferred_element_type=jnp.float32)
    h = h + bl[...]
    m = jnp.mean(h, axis=1, keepdims=True)
    v = jnp.mean((h - m) * (h - m), axis=1, keepdims=True)
    h = (h - m) * lax.rsqrt(v + 1e-5) * g[...] + b[...]
    return jnp.maximum(h, 0.0)


def _tc_layer(sums, cnt2, x_pad, Wl, bl, Wr, g, b):
    def body(p0_r, p1_r, c0_r, c1_r, x_r, wl_r, wr_r, bl_r, g_r, b_r, o_r):
        o_r[...] = _tc_layer_body(p0_r, p1_r, c0_r, c1_r, x_r,
                                  wl_r, wr_r, bl_r, g_r, b_r)

    row_spec = pl.BlockSpec((BLK, D), lambda i: (i, 0))
    w_spec = pl.BlockSpec((D, D), lambda i: (0, 0))
    v_spec = pl.BlockSpec((1, D), lambda i: (0, 0))
    return pl.pallas_call(
        body,
        grid=(GRID,),
        in_specs=[
            pl.BlockSpec((BLK, D), lambda i: (i, 0)),
            pl.BlockSpec((BLK, D), lambda i: (i + GRID, 0)),
            pl.BlockSpec((BLK, 1), lambda i: (i, 0)),
            pl.BlockSpec((BLK, 1), lambda i: (i + GRID, 0)),
            row_spec, w_spec, w_spec, v_spec, v_spec, v_spec,
        ],
        out_specs=row_spec,
        out_shape=jax.ShapeDtypeStruct((NPAD, D), jnp.float32),
    )(sums, sums, cnt2, cnt2, x_pad, Wl, Wr,
      bl.reshape(1, D), g.reshape(1, D), b.reshape(1, D))


def _tc_layer_head(sums, cnt2, x_pad, Wl, bl, Wr, g, b, WhP, bhP):
    def body(p0_r, p1_r, c0_r, c1_r, x_r, wl_r, wr_r, bl_r, g_r, b_r,
             wh_r, bh_r, o_r):
        h = _tc_layer_body(p0_r, p1_r, c0_r, c1_r, x_r,
                           wl_r, wr_r, bl_r, g_r, b_r)
        o_r[...] = lax.dot_general(h, wh_r[...], (((1,), (1,)), ((), ())),
                                   precision=lax.Precision.HIGHEST,
                                   preferred_element_type=jnp.float32) + bh_r[...]

    row_spec = pl.BlockSpec((BLK, D), lambda i: (i, 0))
    w_spec = pl.BlockSpec((D, D), lambda i: (0, 0))
    v_spec = pl.BlockSpec((1, D), lambda i: (0, 0))
    return pl.pallas_call(
        body,
        grid=(GRID,),
        in_specs=[
            pl.BlockSpec((BLK, D), lambda i: (i, 0)),
            pl.BlockSpec((BLK, D), lambda i: (i + GRID, 0)),
            pl.BlockSpec((BLK, 1), lambda i: (i, 0)),
            pl.BlockSpec((BLK, 1), lambda i: (i + GRID, 0)),
            row_spec, w_spec, w_spec, v_spec, v_spec, v_spec,
            w_spec, v_spec,
        ],
        out_specs=row_spec,
        out_shape=jax.ShapeDtypeStruct((NPAD, D), jnp.float32),
    )(sums, sums, cnt2, cnt2, x_pad, Wl, Wr,
      bl.reshape(1, D), g.reshape(1, D), b.reshape(1, D), WhP, bhP)


def kernel(x, edge_index, W1l, b1l, W1r, ln1_g, ln1_b,
           W2l, b2l, W2r, ln2_g, ln2_b, Wh, bh):
    src = jnp.concatenate([edge_index[0], jnp.zeros((EPAD - E,), jnp.int32)])
    pad_dst = N + jnp.arange(EPAD - E, dtype=jnp.int32) % (NPAD - N)
    dst = jnp.concatenate([edge_index[1], pad_dst])
    x_pad = jnp.pad(x, ((0, NPAD - N), (0, 0)))

    sums1, cnts = _segment_sum_counts(x_pad, src, dst)
    cnt2 = cnts.reshape(NCORES * NPAD, 1)
    h1 = _tc_layer(sums1, cnt2, x_pad, W1l, b1l, W1r, ln1_g, ln1_b)
    sums2 = _segment_sum_only(h1, src, dst)
    WhP = jnp.zeros((D, D), jnp.float32).at[:OUT].set(Wh)
    bhP = jnp.zeros((1, D), jnp.float32).at[0, :OUT].set(bh)
    out = _tc_layer_head(sums2, cnt2, h1, W2l, b2l, W2r, ln2_g, ln2_b, WhP, bhP)
    return out[:N, :OUT]


# speedup vs baseline: 1.1137x; 1.0146x over previous
import dataclasses
import functools

import jax
import jax.numpy as jnp
from jax import lax
from jax.experimental import pallas as pl
from jax.experimental.pallas import tpu as pltpu
from jax.experimental.pallas import tpu_sc as plsc

N = 10000
E = 320000
D = 128
OUT = 4

NPAD = 10240
EPAD = 327680
NCORES = 2
NSUB = 16
NW = NCORES * NSUB
E_PER_W = EPAD // NW
CHUNK = 64
N_CHUNKS = E_PER_W // CHUNK
ROWS_PER_SUB = NPAD // NSUB
ZROWS = 64


def _sc_mesh():
    return plsc.VectorSubcoreMesh(core_axis_name="c", subcore_axis_name="s")


def _sc_compiler_params():
    cp = pltpu.CompilerParams()
    if "needs_layout_passes" in pltpu.CompilerParams.__dataclass_fields__:
        cp = dataclasses.replace(cp, needs_layout_passes=False)
    return cp


CROWS = NPAD // D
CSUBS = CROWS // 8
NGROUP = CHUNK // 16
IB = 2048
NIB = E_PER_W // IB
NC2 = IB // CHUNK
RING = 4


def _make_segment_sum(with_counts):
    out_type = jax.ShapeDtypeStruct((NCORES * NPAD, D), jnp.float32)
    if with_counts:
        out_type = (out_type,
                    jax.ShapeDtypeStruct((NCORES * CROWS, D), jnp.float32))
    scratch = [
        pltpu.VMEM_SHARED((NPAD, D), jnp.float32),
        pltpu.VMEM((IB,), jnp.int32),
        pltpu.VMEM((IB,), jnp.int32),
        pltpu.VMEM((CHUNK, D), jnp.float32),
        pltpu.VMEM((CHUNK, D), jnp.float32),
        pltpu.VMEM((CHUNK, D), jnp.float32),
        pltpu.VMEM((CHUNK, D), jnp.float32),
        pltpu.VMEM((CHUNK,), jnp.int32),
        pltpu.VMEM((CHUNK,), jnp.int32),
        pltpu.SemaphoreType.DMA,
        pltpu.SemaphoreType.DMA,
        pltpu.SemaphoreType.DMA,
        pltpu.SemaphoreType.DMA,
        pltpu.SemaphoreType.DMA,
    ]
    if with_counts:
        scratch += [
            pltpu.VMEM_SHARED((CROWS, D), jnp.float32),
            pltpu.VMEM((CHUNK, D), jnp.float32),
            pltpu.VMEM((CHUNK,), jnp.int32),
        ]

    def body(refs):
        if with_counts:
            (x_hbm, src_hbm, dst_hbm, zeros_hbm, tidx_hbm, sum_out, cnt_out,
             acc_sh, sblk_v, dblk_v, rows0_v, rows1_v, rows2_v, rows3_v,
             dstc0_v, dstc1_v, sem0, sem1, sem2, sem3, ssem,
             cnt_sh, crows_v, cidx_v) = refs
        else:
            (x_hbm, src_hbm, dst_hbm, zeros_hbm, tidx_hbm, sum_out,
             acc_sh, sblk_v, dblk_v, rows0_v, rows1_v, rows2_v, rows3_v,
             dstc0_v, dstc1_v, sem0, sem1, sem2, sem3, ssem) = refs
        cid = lax.axis_index("c")
        sid = lax.axis_index("s")
        wid = sid * NCORES + cid
        it16 = lax.iota(jnp.int32, 16)
        ones16 = jnp.ones((16,), jnp.float32)
        zeros16 = jnp.zeros((16,), jnp.float32)

        pltpu.sync_copy(zeros_hbm, rows0_v)
        if with_counts:
            pltpu.sync_copy(zeros_hbm, crows_v)

        row0 = sid * ROWS_PER_SUB

        @pl.loop(0, ROWS_PER_SUB, step=ZROWS)
        def _(r):
            pltpu.sync_copy(rows0_v, acc_sh.at[pl.ds(row0 + r, ZROWS)])

        if with_counts:
            @pl.when(sid < CSUBS)
            def _():
                pltpu.sync_copy(rows0_v.at[pl.ds(0, 8)],
                                cnt_sh.at[pl.ds(sid * 8, 8)])

        plsc.subcore_barrier()

        bufs = (rows0_v, rows1_v, rows2_v, rows3_v)
        sems = (sem0, sem1, sem2, sem3)
        dstcs = (dstc0_v, dstc1_v)

        def g_issue(c, buf, sem):
            pltpu.async_copy(
                x_hbm.at[sblk_v.at[pl.ds(c * CHUNK, CHUNK)]], buf, sem)

        def g_wait(buf, sem):
            pltpu.make_async_copy(
                x_hbm.at[sblk_v.at[pl.ds(0, CHUNK)]], buf, sem).wait()

        def s_wait():
            pltpu.make_async_copy(rows0_v, acc_sh.at[dstc0_v], ssem).wait()

        def process(c, buf, p):
            dstc = dstcs[p]
            for g in range(NGROUP):
                d16 = dblk_v[pl.ds(c * CHUNK + g * 16, 16)]
                dstc[pl.ds(g * 16, 16)] = d16
                if with_counts:
                    col = lax.bitwise_and(d16, D - 1)
                    row = it16 + (g * 16)
                    plsc.store_scatter(crows_v, [row, col], ones16)
                    cidx_v[pl.ds(g * 16, 16)] = lax.shift_right_logical(d16, 7)
            s_wait()
            pltpu.async_copy(buf, acc_sh.at[dstc], ssem, add=True)
            if with_counts:
                pltpu.sync_copy(crows_v, cnt_sh.at[cidx_v], add=True)
                for g in range(NGROUP):
                    d16 = dblk_v[pl.ds(c * CHUNK + g * 16, 16)]
                    col = lax.bitwise_and(d16, D - 1)
                    row = it16 + (g * 16)
                    plsc.store_scatter(crows_v, [row, col], zeros16)

        ebase = wid * E_PER_W

        pltpu.sync_copy(tidx_hbm, dstc0_v)
        pltpu.async_copy(rows0_v, acc_sh.at[dstc0_v], ssem, add=True)

        @pl.loop(0, NIB)
        def _(bi):
            boff = ebase + bi * IB
            pltpu.sync_copy(src_hbm.at[pl.ds(boff, IB)], sblk_v)
            pltpu.sync_copy(dst_hbm.at[pl.ds(boff, IB)], dblk_v)

            for j in range(RING - 1):
                g_issue(j, bufs[j], sems[j])

            @pl.loop(0, NC2 - RING, step=RING)
            def _(c0):
                for j in range(RING):
                    g_wait(bufs[j], sems[j])
                    process(c0 + j, bufs[j], j % 2)
                    g_issue(c0 + j + RING - 1,
                            bufs[(j + RING - 1) % RING], sems[(j + RING - 1) % RING])

            c0 = NC2 - RING
            for j in range(RING):
                g_wait(bufs[j], sems[j])
                process(c0 + j, bufs[j], j % 2)
                if c0 + j + RING - 1 < NC2:
                    g_issue(c0 + j + RING - 1,
                            bufs[(j + RING - 1) % RING], sems[(j + RING - 1) % RING])

        s_wait()
        plsc.subcore_barrier()

        out0 = cid * NPAD + row0

        @pl.loop(0, ROWS_PER_SUB, step=ZROWS)
        def _(r):
            pltpu.sync_copy(acc_sh.at[pl.ds(row0 + r, ZROWS)],
                            sum_out.at[pl.ds(out0 + r, ZROWS)])

        if with_counts:
            @pl.when(sid < CSUBS)
            def _():
                pltpu.sync_copy(
                    cnt_sh.at[pl.ds(sid * 8, 8)],
                    cnt_out.at[pl.ds(cid * CROWS + sid * 8, 8)])

    @functools.partial(
        pl.kernel,
        mesh=_sc_mesh(),
        out_type=out_type,
        compiler_params=_sc_compiler_params(),
        scratch_types=scratch,
    )
    def k(*refs):
        body(refs)

    def run(x_pad, src, dst):
        return k(x_pad, src, dst, jnp.zeros((ZROWS, D), jnp.float32),
                 jnp.full((CHUNK,), NPAD - 1, jnp.int32))

    return run


def _segment_sum_counts(x_pad, src, dst):
    return _make_segment_sum(True)(x_pad, src, dst)


def _segment_sum_only(x_pad, src, dst):
    return _make_segment_sum(False)(x_pad, src, dst)


BLK = 512
GRID = NPAD // BLK


def _tc_layer_body(p0, p1, c0, c1, x, wl, wr, bl, g, b):
    s = p0[...] + p1[...]
    c = c0[...] + c1[...]
    mean = s / jnp.maximum(c, 1.0)
    h = lax.dot_general(mean, wl[...], (((1,), (1,)), ((), ())),
                        precision=lax.Precision.HIGHEST,
                        preferred_element_type=jnp.float32)
    h = h + lax.dot_general(x[...], wr[...], (((1,), (1,)), ((), ())),
                            precision=lax.Precision.HIGHEST,
                            preferred_element_type=jnp.float32)
    h = h + bl[...]
    m = jnp.mean(h, axis=1, keepdims=True)
    v = jnp.mean((h - m) * (h - m), axis=1, keepdims=True)
    h = (h - m) * lax.rsqrt(v + 1e-5) * g[...] + b[...]
    return jnp.maximum(h, 0.0)


def _tc_layer(sums, cnt2, x_pad, Wl, bl, Wr, g, b):
    def body(p0_r, p1_r, c0_r, c1_r, x_r, wl_r, wr_r, bl_r, g_r, b_r, o_r):
        o_r[...] = _tc_layer_body(p0_r, p1_r, c0_r, c1_r, x_r,
                                  wl_r, wr_r, bl_r, g_r, b_r)

    row_spec = pl.BlockSpec((BLK, D), lambda i: (i, 0))
    w_spec = pl.BlockSpec((D, D), lambda i: (0, 0))
    v_spec = pl.BlockSpec((1, D), lambda i: (0, 0))
    return pl.pallas_call(
        body,
        grid=(GRID,),
        in_specs=[
            pl.BlockSpec((BLK, D), lambda i: (i, 0)),
            pl.BlockSpec((BLK, D), lambda i: (i + GRID, 0)),
            pl.BlockSpec((BLK, 1), lambda i: (i, 0)),
            pl.BlockSpec((BLK, 1), lambda i: (i + GRID, 0)),
            row_spec, w_spec, w_spec, v_spec, v_spec, v_spec,
        ],
        out_specs=row_spec,
        out_shape=jax.ShapeDtypeStruct((NPAD, D), jnp.float32),
    )(sums, sums, cnt2, cnt2, x_pad, Wl, Wr,
      bl.reshape(1, D), g.reshape(1, D), b.reshape(1, D))


def _tc_layer_head(sums, cnt2, x_pad, Wl, bl, Wr, g, b, WhP, bhP):
    def body(p0_r, p1_r, c0_r, c1_r, x_r, wl_r, wr_r, bl_r, g_r, b_r,
             wh_r, bh_r, o_r):
        h = _tc_layer_body(p0_r, p1_r, c0_r, c1_r, x_r,
                           wl_r, wr_r, bl_r, g_r, b_r)
        o_r[...] = lax.dot_general(h, wh_r[...], (((1,), (1,)), ((), ())),
                                   precision=lax.Precision.HIGHEST,
                                   preferred_element_type=jnp.float32) + bh_r[...]

    row_spec = pl.BlockSpec((BLK, D), lambda i: (i, 0))
    w_spec = pl.BlockSpec((D, D), lambda i: (0, 0))
    v_spec = pl.BlockSpec((1, D), lambda i: (0, 0))
    return pl.pallas_call(
        body,
        grid=(GRID,),
        in_specs=[
            pl.BlockSpec((BLK, D), lambda i: (i, 0)),
            pl.BlockSpec((BLK, D), lambda i: (i + GRID, 0)),
            pl.BlockSpec((BLK, 1), lambda i: (i, 0)),
            pl.BlockSpec((BLK, 1), lambda i: (i + GRID, 0)),
            row_spec, w_spec, w_spec, v_spec, v_spec, v_spec,
            w_spec, v_spec,
        ],
        out_specs=row_spec,
        out_shape=jax.ShapeDtypeStruct((NPAD, D), jnp.float32),
    )(sums, sums, cnt2, cnt2, x_pad, Wl, Wr,
      bl.reshape(1, D), g.reshape(1, D), b.reshape(1, D), WhP, bhP)


def kernel(x, edge_index, W1l, b1l, W1r, ln1_g, ln1_b,
           W2l, b2l, W2r, ln2_g, ln2_b, Wh, bh):
    src = jnp.concatenate([edge_index[0], jnp.zeros((EPAD - E,), jnp.int32)])
    pad_dst = N + jnp.arange(EPAD - E, dtype=jnp.int32) % (NPAD - N)
    dst = jnp.concatenate([edge_index[1], pad_dst])
    x_pad = jnp.pad(x, ((0, NPAD - N), (0, 0)))

    sums1, cnts = _segment_sum_counts(x_pad, src, dst)
    cnt2 = cnts.reshape(NCORES * NPAD, 1)
    h1 = _tc_layer(sums1, cnt2, x_pad, W1l, b1l, W1r, ln1_g, ln1_b)
    sums2 = _segment_sum_only(h1, src, dst)
    WhP = jnp.zeros((D, D), jnp.float32).at[:OUT].set(Wh)
    bhP = jnp.zeros((1, D), jnp.float32).at[0, :OUT].set(bh)
    out = _tc_layer_head(sums2, cnt2, h1, W2l, b2l, W2r, ln2_g, ln2_b, WhP, bhP)
    return out[:N, :OUT]


# speedup vs baseline: 1.1870x; 1.0658x over previous
import dataclasses
import functools

import jax
import jax.numpy as jnp
from jax import lax
from jax.experimental import pallas as pl
from jax.experimental.pallas import tpu as pltpu
from jax.experimental.pallas import tpu_sc as plsc

N = 10000
E = 320000
D = 128
OUT = 4

NPAD = 10240
EPAD = 327680
NCORES = 2
NSUB = 16
NW = NCORES * NSUB
E_PER_W = EPAD // NW
CHUNK = 64
N_CHUNKS = E_PER_W // CHUNK
ROWS_PER_SUB = NPAD // NSUB
ZROWS = 64


def _sc_mesh():
    return plsc.VectorSubcoreMesh(core_axis_name="c", subcore_axis_name="s")


def _sc_compiler_params():
    cp = pltpu.CompilerParams()
    if "needs_layout_passes" in pltpu.CompilerParams.__dataclass_fields__:
        cp = dataclasses.replace(cp, needs_layout_passes=False)
    return cp


CROWS = NPAD // D
CSUBS = CROWS // 8
NGROUP = CHUNK // 16
IB = 2048
NIB = E_PER_W // IB
NC2 = IB // CHUNK
RING = 4
NIB0 = 7
NIB1 = 3


def _make_segment_sum(with_counts):
    out_type = jax.ShapeDtypeStruct((NCORES * NPAD, D), jnp.float32)
    if with_counts:
        out_type = (out_type,
                    jax.ShapeDtypeStruct((NCORES * CROWS, D), jnp.float32))
    scratch = [
        pltpu.VMEM_SHARED((NPAD, D), jnp.float32),
        pltpu.VMEM((IB,), jnp.int32),
        pltpu.VMEM((IB,), jnp.int32),
        pltpu.VMEM((CHUNK, D), jnp.float32),
        pltpu.VMEM((CHUNK, D), jnp.float32),
        pltpu.VMEM((CHUNK, D), jnp.float32),
        pltpu.VMEM((CHUNK, D), jnp.float32),
        pltpu.VMEM((CHUNK,), jnp.int32),
        pltpu.VMEM((CHUNK,), jnp.int32),
        pltpu.SemaphoreType.DMA,
        pltpu.SemaphoreType.DMA,
        pltpu.SemaphoreType.DMA,
        pltpu.SemaphoreType.DMA,
        pltpu.SemaphoreType.DMA,
    ]
    if with_counts:
        scratch += [
            pltpu.VMEM_SHARED((CROWS, D), jnp.float32),
            pltpu.VMEM((CHUNK, D), jnp.float32),
            pltpu.VMEM((CHUNK,), jnp.int32),
        ]

    def body(refs):
        if with_counts:
            (x_hbm, src_hbm, dst_hbm, zeros_hbm, tidx_hbm, sum_out, cnt_out,
             acc_sh, sblk_v, dblk_v, rows0_v, rows1_v, rows2_v, rows3_v,
             dstc0_v, dstc1_v, sem0, sem1, sem2, sem3, ssem,
             cnt_sh, crows_v, cidx_v) = refs
        else:
            (x_hbm, src_hbm, dst_hbm, zeros_hbm, tidx_hbm, sum_out,
             acc_sh, sblk_v, dblk_v, rows0_v, rows1_v, rows2_v, rows3_v,
             dstc0_v, dstc1_v, sem0, sem1, sem2, sem3, ssem) = refs
        cid = lax.axis_index("c")
        sid = lax.axis_index("s")
        wid = sid * NCORES + cid
        it16 = lax.iota(jnp.int32, 16)
        ones16 = jnp.ones((16,), jnp.float32)
        zeros16 = jnp.zeros((16,), jnp.float32)

        pltpu.sync_copy(zeros_hbm, rows0_v)
        if with_counts:
            pltpu.sync_copy(zeros_hbm, crows_v)

        row0 = sid * ROWS_PER_SUB

        @pl.loop(0, ROWS_PER_SUB, step=ZROWS)
        def _(r):
            pltpu.sync_copy(rows0_v, acc_sh.at[pl.ds(row0 + r, ZROWS)])

        if with_counts:
            @pl.when(sid < CSUBS)
            def _():
                pltpu.sync_copy(rows0_v.at[pl.ds(0, 8)],
                                cnt_sh.at[pl.ds(sid * 8, 8)])

        plsc.subcore_barrier()

        bufs = (rows0_v, rows1_v, rows2_v, rows3_v)
        sems = (sem0, sem1, sem2, sem3)
        dstcs = (dstc0_v, dstc1_v)

        def g_issue(c, buf, sem):
            pltpu.async_copy(
                x_hbm.at[sblk_v.at[pl.ds(c * CHUNK, CHUNK)]], buf, sem)

        def g_wait(buf, sem):
            pltpu.make_async_copy(
                x_hbm.at[sblk_v.at[pl.ds(0, CHUNK)]], buf, sem).wait()

        def s_wait():
            pltpu.make_async_copy(rows0_v, acc_sh.at[dstc0_v], ssem).wait()

        def process(c, buf, p):
            dstc = dstcs[p]
            for g in range(NGROUP):
                d16 = dblk_v[pl.ds(c * CHUNK + g * 16, 16)]
                dstc[pl.ds(g * 16, 16)] = d16
                if with_counts:
                    col = lax.bitwise_and(d16, D - 1)
                    row = it16 + (g * 16)
                    plsc.store_scatter(crows_v, [row, col], ones16)
                    cidx_v[pl.ds(g * 16, 16)] = lax.shift_right_logical(d16, 7)
            s_wait()
            pltpu.async_copy(buf, acc_sh.at[dstc], ssem, add=True)
            if with_counts:
                pltpu.sync_copy(crows_v, cnt_sh.at[cidx_v], add=True)
                for g in range(NGROUP):
                    d16 = dblk_v[pl.ds(c * CHUNK + g * 16, 16)]
                    col = lax.bitwise_and(d16, D - 1)
                    row = it16 + (g * 16)
                    plsc.store_scatter(crows_v, [row, col], zeros16)

        nib = jnp.where(cid == 0, NIB0, NIB1)
        ebase = jnp.where(cid == 0, sid * (NIB0 * IB),
                          NSUB * NIB0 * IB + sid * (NIB1 * IB))

        pltpu.sync_copy(tidx_hbm, dstc0_v)
        pltpu.async_copy(rows0_v, acc_sh.at[dstc0_v], ssem, add=True)

        @pl.loop(0, nib)
        def _(bi):
            boff = ebase + bi * IB
            pltpu.sync_copy(src_hbm.at[pl.ds(boff, IB)], sblk_v)
            pltpu.sync_copy(dst_hbm.at[pl.ds(boff, IB)], dblk_v)

            for j in range(RING - 1):
                g_issue(j, bufs[j], sems[j])

            @pl.loop(0, NC2 - RING, step=RING)
            def _(c0):
                for j in range(RING):
                    g_wait(bufs[j], sems[j])
                    process(c0 + j, bufs[j], j % 2)
                    g_issue(c0 + j + RING - 1,
                            bufs[(j + RING - 1) % RING], sems[(j + RING - 1) % RING])

            c0 = NC2 - RING
            for j in range(RING):
                g_wait(bufs[j], sems[j])
                process(c0 + j, bufs[j], j % 2)
                if c0 + j + RING - 1 < NC2:
                    g_issue(c0 + j + RING - 1,
                            bufs[(j + RING - 1) % RING], sems[(j + RING - 1) % RING])

        s_wait()
        plsc.subcore_barrier()

        out0 = cid * NPAD + row0

        @pl.loop(0, ROWS_PER_SUB, step=ZROWS)
        def _(r):
            pltpu.sync_copy(acc_sh.at[pl.ds(row0 + r, ZROWS)],
                            sum_out.at[pl.ds(out0 + r, ZROWS)])

        if with_counts:
            @pl.when(sid < CSUBS)
            def _():
                pltpu.sync_copy(
                    cnt_sh.at[pl.ds(sid * 8, 8)],
                    cnt_out.at[pl.ds(cid * CROWS + sid * 8, 8)])

    @functools.partial(
        pl.kernel,
        mesh=_sc_mesh(),
        out_type=out_type,
        compiler_params=_sc_compiler_params(),
        scratch_types=scratch,
    )
    def k(*refs):
        body(refs)

    def run(x_pad, src, dst):
        return k(x_pad, src, dst, jnp.zeros((ZROWS, D), jnp.float32),
                 jnp.full((CHUNK,), NPAD - 1, jnp.int32))

    return run


def _segment_sum_counts(x_pad, src, dst):
    return _make_segment_sum(True)(x_pad, src, dst)


def _segment_sum_only(x_pad, src, dst):
    return _make_segment_sum(False)(x_pad, src, dst)


BLK = 512
GRID = NPAD // BLK


def _tc_layer_body(p0, p1, c0, c1, x, wl, wr, bl, g, b):
    s = p0[...] + p1[...]
    c = c0[...] + c1[...]
    mean = s / jnp.maximum(c, 1.0)
    h = lax.dot_general(mean, wl[...], (((1,), (1,)), ((), ())),
                        precision=lax.Precision.HIGHEST,
                        preferred_element_type=jnp.float32)
    h = h + lax.dot_general(x[...], wr[...], (((1,), (1,)), ((), ())),
                            precision=lax.Precision.HIGHEST,
                            preferred_element_type=jnp.float32)
    h = h + bl[...]
    m = jnp.mean(h, axis=1, keepdims=True)
    v = jnp.mean((h - m) * (h - m), axis=1, keepdims=True)
    h = (h - m) * lax.rsqrt(v + 1e-5) * g[...] + b[...]
    return jnp.maximum(h, 0.0)


def _tc_layer(sums, cnt2, x_pad, Wl, bl, Wr, g, b):
    def body(p0_r, p1_r, c0_r, c1_r, x_r, wl_r, wr_r, bl_r, g_r, b_r, o_r):
        o_r[...] = _tc_layer_body(p0_r, p1_r, c0_r, c1_r, x_r,
                                  wl_r, wr_r, bl_r, g_r, b_r)

    row_spec = pl.BlockSpec((BLK, D), lambda i: (i, 0))
    w_spec = pl.BlockSpec((D, D), lambda i: (0, 0))
    v_spec = pl.BlockSpec((1, D), lambda i: (0, 0))
    return pl.pallas_call(
        body,
        grid=(GRID,),
        in_specs=[
            pl.BlockSpec((BLK, D), lambda i: (i, 0)),
            pl.BlockSpec((BLK, D), lambda i: (i + GRID, 0)),
            pl.BlockSpec((BLK, 1), lambda i: (i, 0)),
            pl.BlockSpec((BLK, 1), lambda i: (i + GRID, 0)),
            row_spec, w_spec, w_spec, v_spec, v_spec, v_spec,
        ],
        out_specs=row_spec,
        out_shape=jax.ShapeDtypeStruct((NPAD, D), jnp.float32),
    )(sums, sums, cnt2, cnt2, x_pad, Wl, Wr,
      bl.reshape(1, D), g.reshape(1, D), b.reshape(1, D))


def _tc_layer_head(sums, cnt2, x_pad, Wl, bl, Wr, g, b, WhP, bhP):
    def body(p0_r, p1_r, c0_r, c1_r, x_r, wl_r, wr_r, bl_r, g_r, b_r,
             wh_r, bh_r, o_r):
        h = _tc_layer_body(p0_r, p1_r, c0_r, c1_r, x_r,
                           wl_r, wr_r, bl_r, g_r, b_r)
        o_r[...] = lax.dot_general(h, wh_r[...], (((1,), (1,)), ((), ())),
                                   precision=lax.Precision.HIGHEST,
                                   preferred_element_type=jnp.float32) + bh_r[...]

    row_spec = pl.BlockSpec((BLK, D), lambda i: (i, 0))
    w_spec = pl.BlockSpec((D, D), lambda i: (0, 0))
    v_spec = pl.BlockSpec((1, D), lambda i: (0, 0))
    return pl.pallas_call(
        body,
        grid=(GRID,),
        in_specs=[
            pl.BlockSpec((BLK, D), lambda i: (i, 0)),
            pl.BlockSpec((BLK, D), lambda i: (i + GRID, 0)),
            pl.BlockSpec((BLK, 1), lambda i: (i, 0)),
            pl.BlockSpec((BLK, 1), lambda i: (i + GRID, 0)),
            row_spec, w_spec, w_spec, v_spec, v_spec, v_spec,
            w_spec, v_spec,
        ],
        out_specs=row_spec,
        out_shape=jax.ShapeDtypeStruct((NPAD, D), jnp.float32),
    )(sums, sums, cnt2, cnt2, x_pad, Wl, Wr,
      bl.reshape(1, D), g.reshape(1, D), b.reshape(1, D), WhP, bhP)


def kernel(x, edge_index, W1l, b1l, W1r, ln1_g, ln1_b,
           W2l, b2l, W2r, ln2_g, ln2_b, Wh, bh):
    src = jnp.concatenate([edge_index[0], jnp.zeros((EPAD - E,), jnp.int32)])
    pad_dst = N + jnp.arange(EPAD - E, dtype=jnp.int32) % (NPAD - N)
    dst = jnp.concatenate([edge_index[1], pad_dst])
    x_pad = jnp.pad(x, ((0, NPAD - N), (0, 0)))

    sums1, cnts = _segment_sum_counts(x_pad, src, dst)
    cnt2 = cnts.reshape(NCORES * NPAD, 1)
    h1 = _tc_layer(sums1, cnt2, x_pad, W1l, b1l, W1r, ln1_g, ln1_b)
    sums2 = _segment_sum_only(h1, src, dst)
    WhP = jnp.zeros((D, D), jnp.float32).at[:OUT].set(Wh)
    bhP = jnp.zeros((1, D), jnp.float32).at[0, :OUT].set(bh)
    out = _tc_layer_head(sums2, cnt2, h1, W2l, b2l, W2r, ln2_g, ln2_b, WhP, bhP)
    return out[:N, :OUT]
